# baseline (speedup 1.0000x reference)
import functools

import jax
import jax.numpy as jnp
import numpy as np
from jax import lax
from jax.experimental import pallas as pl
from jax.experimental.pallas import tpu as pltpu
from jax.experimental.pallas import tpu_sc as plsc

_MASK_PCT = 0.75

_NC = 2
_NS = 16
_NW = _NC * _NS

_MK = 96
_UK = 32


def _threefry2x32(k1, k2, x1, x2):
    def rotl(x, d):
        return ((x << np.uint32(d)) | (x >> np.uint32(32 - d))).astype(np.uint32)

    x = [np.asarray(x1, np.uint32).copy(), np.asarray(x2, np.uint32).copy()]
    ks = [np.uint32(k1), np.uint32(k2),
          np.uint32(k1) ^ np.uint32(k2) ^ np.uint32(0x1BD11BDA)]
    rot0, rot1 = (13, 15, 26, 6), (17, 29, 16, 24)

    def rounds(x, rots):
        for r in rots:
            x[0] = (x[0] + x[1]).astype(np.uint32)
            x[1] = x[0] ^ rotl(x[1], r)
        return x

    x[0] = (x[0] + ks[0]).astype(np.uint32)
    x[1] = (x[1] + ks[1]).astype(np.uint32)
    for i, (rots, kk) in enumerate(
        [(rot0, (1, 2)), (rot1, (2, 0)), (rot0, (0, 1)),
         (rot1, (1, 2)), (rot0, (2, 0))]
    ):
        x = rounds(x, rots)
        x[0] = (x[0] + ks[kk[0]]).astype(np.uint32)
        x[1] = (x[1] + ks[kk[1]] + np.uint32(i + 1)).astype(np.uint32)
    return x


def _tf_counts(k1, k2, n):
    hi = np.zeros(n, dtype=np.uint32)
    lo = np.arange(n, dtype=np.uint32)
    return _threefry2x32(k1, k2, hi, lo)


@functools.lru_cache(maxsize=None)
def _index_plan(B, N):
    M = int(_MASK_PCT * N)

    b1, b2 = _tf_counts(np.uint32(0), np.uint32(42), B)
    arange = np.arange(N, dtype=np.int32)
    perms = []
    for kb1, kb2 in zip(b1, b2):
        s1, s2 = _tf_counts(kb1, kb2, 2)
        r1, r2 = _tf_counts(s1[1], s2[1], N)
        bits = r1 ^ r2
        perms.append(arange[np.argsort(bits, kind="stable")])
    perms = np.stack(perms)
    masked = perms[:, :M].astype(np.int32)
    unmasked = perms[:, M:].astype(np.int32)

    rows = np.arange(B, dtype=np.int32)[:, None] * N
    mflat = (masked + rows).reshape(-1)
    uflat = (unmasked + rows).reshape(-1)

    def chunked(flat, k):
        per_w = -(-flat.size // (_NW * k)) * k
        pad = _NW * per_w - flat.size
        flat = np.concatenate([flat, flat[:pad]])
        return np.ascontiguousarray(flat.reshape(_NW, per_w // k, k))

    return masked, unmasked, chunked(mflat, _MK), chunked(uflat, _UK)


def _sc_body(n_mc, n_uc, x_hbm, embrep_hbm, midx_hbm, uidx_hbm, out_hbm,
             midx_v, uidx_v, emb_v, buf0, buf1,
             msem, gsem0, gsem1, ssem0, ssem1):
    w = lax.axis_index("s") * _NC + lax.axis_index("c")
    h_mi = pltpu.async_copy(midx_hbm.at[w], midx_v, gsem0)
    h_ui = pltpu.async_copy(uidx_hbm.at[w], uidx_v, gsem1)
    h_em = pltpu.async_copy(embrep_hbm, emb_v, ssem0)
    h_mi.wait()
    h_ui.wait()
    h_em.wait()

    bufs = (buf0, buf1)
    gsems = (gsem0, gsem1)
    ssems = (ssem0, ssem1)

    gh = [None, None]
    for ci in range(min(2, n_uc)):
        gh[ci] = pltpu.async_copy(x_hbm.at[uidx_v.at[ci]], bufs[ci], gsems[ci])
    prime = min(2, n_mc)
    mh = [
        pltpu.async_copy(emb_v, out_hbm.at[midx_v.at[ci]], msem)
        for ci in range(prime)
    ]
    sh = [None, None]
    for ci in range(n_uc):
        b = ci % 2
        gh[b].wait()
        sh[b] = pltpu.async_copy(bufs[b], out_hbm.at[uidx_v.at[ci]], ssems[b])
        if ci + 2 < n_uc:
            sh[b].wait()
            gh[b] = pltpu.async_copy(
                x_hbm.at[uidx_v.at[ci + 2]], bufs[b], gsems[b]
            )

    mh += [
        pltpu.async_copy(emb_v, out_hbm.at[midx_v.at[ci]], msem)
        for ci in range(prime, n_mc)
    ]

    for b in range(2):
        if sh[b] is not None:
            sh[b].wait()
    for h in mh:
        h.wait()


def kernel(patch_embeddings, encoder_mask_emb):
    B, N, D = patch_embeddings.shape

    masked_np, unmasked_np, mchunks_np, uchunks_np = _index_plan(B, N)
    masked_indices = jnp.asarray(masked_np)
    unmasked_indices = jnp.asarray(unmasked_np)
    mchunks = jnp.asarray(mchunks_np)
    uchunks = jnp.asarray(uchunks_np)
    n_mc = mchunks_np.shape[1]
    n_uc = uchunks_np.shape[1]

    x2 = patch_embeddings.reshape(B * N, D)
    embrep = jnp.broadcast_to(encoder_mask_emb, (_MK, D))

    mesh = plsc.VectorSubcoreMesh(core_axis_name="c", subcore_axis_name="s")
    sc = pl.kernel(
        functools.partial(_sc_body, n_mc, n_uc),
        mesh=mesh,
        out_type=jax.ShapeDtypeStruct((B * N, D), patch_embeddings.dtype),
        scratch_types=[
            pltpu.VMEM((n_mc, _MK), jnp.int32),
            pltpu.VMEM((n_uc, _UK), jnp.int32),
            pltpu.VMEM((_MK, D), jnp.float32),
            pltpu.VMEM((_UK, D), jnp.float32),
            pltpu.VMEM((_UK, D), jnp.float32),
            pltpu.SemaphoreType.DMA,
            pltpu.SemaphoreType.DMA,
            pltpu.SemaphoreType.DMA,
            pltpu.SemaphoreType.DMA,
            pltpu.SemaphoreType.DMA,
        ],
    )
    out = sc(x2, embrep, mchunks, uchunks).reshape(B, N, D)

    return out, masked_indices, unmasked_indices


# speedup vs baseline: 1.0520x; 1.0520x over previous
import functools

import jax
import jax.numpy as jnp
import numpy as np
from jax import lax
from jax.experimental import pallas as pl
from jax.experimental.pallas import tpu as pltpu
from jax.experimental.pallas import tpu_sc as plsc

_MASK_PCT = 0.75

_NC = 2
_NS = 16
_NW = _NC * _NS

_MK = 48
_UK = 48


def _threefry2x32(k1, k2, x1, x2):
    def rotl(x, d):
        return ((x << np.uint32(d)) | (x >> np.uint32(32 - d))).astype(np.uint32)

    x = [np.asarray(x1, np.uint32).copy(), np.asarray(x2, np.uint32).copy()]
    ks = [np.uint32(k1), np.uint32(k2),
          np.uint32(k1) ^ np.uint32(k2) ^ np.uint32(0x1BD11BDA)]
    rot0, rot1 = (13, 15, 26, 6), (17, 29, 16, 24)

    def rounds(x, rots):
        for r in rots:
            x[0] = (x[0] + x[1]).astype(np.uint32)
            x[1] = x[0] ^ rotl(x[1], r)
        return x

    x[0] = (x[0] + ks[0]).astype(np.uint32)
    x[1] = (x[1] + ks[1]).astype(np.uint32)
    for i, (rots, kk) in enumerate(
        [(rot0, (1, 2)), (rot1, (2, 0)), (rot0, (0, 1)),
         (rot1, (1, 2)), (rot0, (2, 0))]
    ):
        x = rounds(x, rots)
        x[0] = (x[0] + ks[kk[0]]).astype(np.uint32)
        x[1] = (x[1] + ks[kk[1]] + np.uint32(i + 1)).astype(np.uint32)
    return x


def _tf_counts(k1, k2, n):
    hi = np.zeros(n, dtype=np.uint32)
    lo = np.arange(n, dtype=np.uint32)
    return _threefry2x32(k1, k2, hi, lo)


@functools.lru_cache(maxsize=None)
def _index_plan(B, N):
    M = int(_MASK_PCT * N)

    b1, b2 = _tf_counts(np.uint32(0), np.uint32(42), B)
    arange = np.arange(N, dtype=np.int32)
    perms = []
    for kb1, kb2 in zip(b1, b2):
        s1, s2 = _tf_counts(kb1, kb2, 2)
        r1, r2 = _tf_counts(s1[1], s2[1], N)
        bits = r1 ^ r2
        perms.append(arange[np.argsort(bits, kind="stable")])
    perms = np.stack(perms)
    masked = perms[:, :M].astype(np.int32)
    unmasked = perms[:, M:].astype(np.int32)

    rows = np.arange(B, dtype=np.int32)[:, None] * N
    mflat = (masked + rows).reshape(-1)
    uflat = (unmasked + rows).reshape(-1)

    def chunked(flat, k):
        per_w = -(-flat.size // (_NW * k)) * k
        pad = _NW * per_w - flat.size
        flat = np.concatenate([flat, flat[:pad]])
        return np.ascontiguousarray(flat.reshape(_NW, per_w // k, k))

    return masked, unmasked, chunked(mflat, _MK), chunked(uflat, _UK)


def _sc_body(n_mc, n_uc, x_hbm, embrep_hbm, midx_hbm, uidx_hbm, out_hbm,
             midx_v, uidx_v, emb_v, buf0, buf1,
             msem, gsem0, gsem1, ssem0, ssem1):
    w = lax.axis_index("s") * _NC + lax.axis_index("c")
    h_mi = pltpu.async_copy(midx_hbm.at[w], midx_v, gsem0)
    h_ui = pltpu.async_copy(uidx_hbm.at[w], uidx_v, gsem1)
    h_em = pltpu.async_copy(embrep_hbm, emb_v, ssem0)
    h_mi.wait()
    h_ui.wait()
    h_em.wait()

    bufs = (buf0, buf1)
    gsems = (gsem0, gsem1)
    ssems = (ssem0, ssem1)

    gh = [None, None]
    for ci in range(min(2, n_uc)):
        gh[ci] = pltpu.async_copy(x_hbm.at[uidx_v.at[ci]], bufs[ci], gsems[ci])
    prime = min(4, n_mc)
    mh = [
        pltpu.async_copy(emb_v, out_hbm.at[midx_v.at[ci]], msem)
        for ci in range(prime)
    ]
    sh = [None, None]
    for ci in range(n_uc):
        b = ci % 2
        gh[b].wait()
        sh[b] = pltpu.async_copy(bufs[b], out_hbm.at[uidx_v.at[ci]], ssems[b])
        if ci + 2 < n_uc:
            sh[b].wait()
            gh[b] = pltpu.async_copy(
                x_hbm.at[uidx_v.at[ci + 2]], bufs[b], gsems[b]
            )

    mh += [
        pltpu.async_copy(emb_v, out_hbm.at[midx_v.at[ci]], msem)
        for ci in range(prime, n_mc)
    ]

    for b in range(2):
        if sh[b] is not None:
            sh[b].wait()
    for h in mh:
        h.wait()


def kernel(patch_embeddings, encoder_mask_emb):
    B, N, D = patch_embeddings.shape

    masked_np, unmasked_np, mchunks_np, uchunks_np = _index_plan(B, N)
    masked_indices = jnp.asarray(masked_np)
    unmasked_indices = jnp.asarray(unmasked_np)
    mchunks = jnp.asarray(mchunks_np)
    uchunks = jnp.asarray(uchunks_np)
    n_mc = mchunks_np.shape[1]
    n_uc = uchunks_np.shape[1]

    x2 = patch_embeddings.reshape(B * N, D)
    embrep = jnp.broadcast_to(encoder_mask_emb, (_MK, D))

    mesh = plsc.VectorSubcoreMesh(core_axis_name="c", subcore_axis_name="s")
    sc = pl.kernel(
        functools.partial(_sc_body, n_mc, n_uc),
        mesh=mesh,
        out_type=jax.ShapeDtypeStruct((B * N, D), patch_embeddings.dtype),
        scratch_types=[
            pltpu.VMEM((n_mc, _MK), jnp.int32),
            pltpu.VMEM((n_uc, _UK), jnp.int32),
            pltpu.VMEM((_MK, D), jnp.float32),
            pltpu.VMEM((_UK, D), jnp.float32),
            pltpu.VMEM((_UK, D), jnp.float32),
            pltpu.SemaphoreType.DMA,
            pltpu.SemaphoreType.DMA,
            pltpu.SemaphoreType.DMA,
            pltpu.SemaphoreType.DMA,
            pltpu.SemaphoreType.DMA,
        ],
    )
    out = sc(x2, embrep, mchunks, uchunks).reshape(B, N, D)

    return out, masked_indices, unmasked_indices
